# baseline (speedup 1.0000x reference)
import functools

import jax
import jax.numpy as jnp
from jax import lax
from jax.experimental import pallas as pl
from jax.experimental.pallas import tpu as pltpu
from jax.experimental.pallas import tpu_sc as plsc

N_NODES_K = 10000
N_EDGES_K = 160000
D = 256

NC = 2
NS = 16
NW = NC * NS
W_WIN = 313
ACC_ROWS = 320
TRASH = 316
SCAN = 1024
NSCAN = 158
NCH_PAD = 160
E_PAD = NCH_PAD * SCAN
G = 64
FLUSH_AT = 1024
CAP = 2560
DST_SENTINEL = 1 << 30

_sc_mesh = plsc.VectorSubcoreMesh(core_axis_name="c", subcore_axis_name="s")


@functools.partial(
    pl.kernel,
    out_type=[
        jax.ShapeDtypeStruct((NW, ACC_ROWS, D), jnp.float32),
        jax.ShapeDtypeStruct((NW, ACC_ROWS), jnp.float32),
    ],
    mesh=_sc_mesh,
    scratch_types=[
        pltpu.VMEM((ACC_ROWS, D), jnp.float32),
        pltpu.VMEM((G, D), jnp.float32),
        pltpu.VMEM((G, D), jnp.float32),
        pltpu.VMEM((SCAN,), jnp.int32),
        pltpu.VMEM((SCAN,), jnp.int32),
        pltpu.VMEM((SCAN,), jnp.int32),
        pltpu.VMEM((SCAN,), jnp.int32),
        pltpu.VMEM((CAP,), jnp.int32),
        pltpu.VMEM((CAP,), jnp.int32),
        pltpu.VMEM((G,), jnp.int32),
        pltpu.VMEM((G,), jnp.float32),
        pltpu.VMEM((ACC_ROWS,), jnp.float32),
        pltpu.VMEM_SHARED((NS * ACC_ROWS,), jnp.float32),
        pltpu.SemaphoreType.DMA,
        pltpu.SemaphoreType.DMA,
        pltpu.SemaphoreType.DMA,
        pltpu.SemaphoreType.DMA,
    ],
    compiler_params=pltpu.CompilerParams(needs_layout_passes=False),
)
def _sc_aggregate(feat_hbm, src_hbm, dst_hbm, sum_hbm, deg_hbm,
                  acc, rowsA, rowsB, dstA, srcA, dstB, srcB,
                  csrc, cld, cidx, ones, degf, degsp,
                  semA, semB, semGA, semGB):
    c = lax.axis_index("c")
    s = lax.axis_index("s")
    w = s * NC + c
    base = w * W_WIN
    dbase = s * ACC_ROWS
    zero16 = jnp.zeros((16,), jnp.float32)
    one16 = jnp.ones((16,), jnp.float32)
    trash16 = jnp.full((16,), TRASH, jnp.int32)
    zero16i = jnp.zeros((16,), jnp.int32)
    iota16 = lax.iota(jnp.int32, 16)

    def _zrow(i, carry):
        for j in range(D // 16):
            acc[i, pl.ds(j * 16, 16)] = zero16
        return carry
    lax.fori_loop(0, ACC_ROWS, _zrow, 0)
    for j in range(ACC_ROWS // 16):
        degf[pl.ds(j * 16, 16)] = zero16
    for j in range(G // 16):
        ones[pl.ds(j * 16, 16)] = one16
    def _zc(i, carry):
        csrc[pl.ds(i * 16, 16)] = zero16i
        return carry
    lax.fori_loop(0, CAP // 16, _zc, 0)
    pltpu.sync_copy(degf, degsp.at[pl.ds(dbase, ACC_ROWS)])

    def _wait_scan(sem, dbuf, sbuf):
        pltpu.make_async_copy(dst_hbm.at[pl.ds(0, SCAN)], dbuf, sem).wait()
        pltpu.make_async_copy(src_hbm.at[pl.ds(0, SCAN)], sbuf, sem).wait()

    def _wait_rows(sem, rbuf):
        pltpu.make_async_copy(feat_hbm.at[pl.ds(0, G)], rbuf, sem).wait()

    def _flush(n):
        a0 = (n // 16) * 16
        keep = iota16 < (n - a0)
        csrc[pl.ds(a0, 16)] = jnp.where(keep, csrc[pl.ds(a0, 16)], zero16i)
        cld[pl.ds(a0, 16)] = jnp.where(keep, cld[pl.ds(a0, 16)], trash16)
        for t in range(1, G // 16):
            csrc[pl.ds(a0 + 16 * t, 16)] = zero16i
            cld[pl.ds(a0 + 16 * t, 16)] = trash16
        nch = (n + G - 1) // G
        nh = (nch + 1) // 2

        def _chunk(p0, rbuf):
            for j in range(G // 16):
                cidx[pl.ds(j * 16, 16)] = cld[pl.ds(p0 + j * 16, 16)] + dbase
            pltpu.sync_copy(ones, degsp.at[cidx], add=True)

            def _edge16(e16, carry2):
                rv = cld[pl.ds(p0 + e16 * 16, 16)]
                for k in range(16):
                    r = rv[k]
                    e = e16 * 16 + k
                    for j in range(D // 16):
                        plsc.addupdate(acc.at[r, pl.ds(j * 16, 16)],
                                       rbuf[e, pl.ds(j * 16, 16)])
                return carry2
            lax.fori_loop(0, G // 16, _edge16, 0)

        pltpu.async_copy(feat_hbm.at[csrc.at[pl.ds(0, G)]], rowsA, semGA)

        def _pair(g2, carry):
            pA = (2 * g2) * G
            pB = pA + G
            pltpu.async_copy(feat_hbm.at[csrc.at[pl.ds(pB, G)]], rowsB, semGB)
            _wait_rows(semGA, rowsA)
            pl.when(2 * g2 < nch)(lambda: _chunk(pA, rowsA))
            pltpu.async_copy(feat_hbm.at[csrc.at[pl.ds(pB + G, G)]],
                             rowsA, semGA)
            _wait_rows(semGB, rowsB)
            pl.when(2 * g2 + 1 < nch)(lambda: _chunk(pB, rowsB))
            return carry
        lax.fori_loop(0, nh, _pair, 0)
        _wait_rows(semGA, rowsA)

    def _proc(dbuf, sbuf, cnt):
        def _step(i, cnt2):
            d = dbuf[pl.ds(i * 16, 16)]
            ld = d - base
            m = plsc.bitcast(ld, jnp.uint32) < jnp.uint32(W_WIN)
            incl = plsc.cumsum(jnp.where(m, 1, 0).astype(jnp.int32))
            pos = cnt2 + incl - 1
            plsc.store_scatter(csrc, [pos], sbuf[pl.ds(i * 16, 16)], mask=m)
            plsc.store_scatter(cld, [pos], ld, mask=m)
            return cnt2 + jnp.max(incl)
        cnt = lax.fori_loop(0, SCAN // 16, _step, cnt)
        do_flush = cnt >= FLUSH_AT
        pl.when(do_flush)(lambda: _flush(cnt))
        return jnp.where(do_flush, 0, cnt)

    pltpu.async_copy(dst_hbm.at[pl.ds(0, SCAN)], dstA, semA)
    pltpu.async_copy(src_hbm.at[pl.ds(0, SCAN)], srcA, semA)

    def _scan2(h, cnt):
        offB = (2 * h + 1) * SCAN
        offA2 = (2 * h + 2) * SCAN
        pltpu.async_copy(dst_hbm.at[pl.ds(offB, SCAN)], dstB, semB)
        pltpu.async_copy(src_hbm.at[pl.ds(offB, SCAN)], srcB, semB)
        _wait_scan(semA, dstA, srcA)
        cnt = _proc(dstA, srcA, cnt)
        pltpu.async_copy(dst_hbm.at[pl.ds(offA2, SCAN)], dstA, semA)
        pltpu.async_copy(src_hbm.at[pl.ds(offA2, SCAN)], srcA, semA)
        _wait_scan(semB, dstB, srcB)
        cnt = _proc(dstB, srcB, cnt)
        return cnt
    cnt = lax.fori_loop(0, NSCAN // 2, _scan2, jnp.int32(0))
    _wait_scan(semA, dstA, srcA)
    pl.when(cnt > 0)(lambda: _flush(cnt))

    pltpu.sync_copy(acc, sum_hbm.at[w])
    pltpu.sync_copy(degsp.at[pl.ds(dbase, ACC_ROWS)], degf)
    pltpu.sync_copy(degf, deg_hbm.at[w])


def _tc_body(feat_ref, sum_ref, deg_ref, wst_ref, wnt_ref, b_ref, out_ref):
    rcp = 1.0 / jnp.maximum(deg_ref[...], 1.0)
    h = sum_ref[...] * rcp
    out_ref[...] = (
        jnp.dot(feat_ref[...], wst_ref[...], preferred_element_type=jnp.float32)
        + jnp.dot(h, wnt_ref[...], preferred_element_type=jnp.float32)
        + b_ref[...]
    )


_BLK = 200
_tc_combine = pl.pallas_call(
    _tc_body,
    grid=(N_NODES_K // _BLK,),
    in_specs=[
        pl.BlockSpec((_BLK, D), lambda i: (i, 0)),
        pl.BlockSpec((_BLK, D), lambda i: (i, 0)),
        pl.BlockSpec((_BLK, 1), lambda i: (i, 0)),
        pl.BlockSpec((D, D), lambda i: (0, 0)),
        pl.BlockSpec((D, D), lambda i: (0, 0)),
        pl.BlockSpec((1, D), lambda i: (0, 0)),
    ],
    out_specs=pl.BlockSpec((_BLK, D), lambda i: (i, 0)),
    out_shape=jax.ShapeDtypeStruct((N_NODES_K, D), jnp.float32),
)


@jax.jit
def kernel(feat, edge_index, W_self, W_neigh, b):
    npad = E_PAD - N_EDGES_K
    srcp = jnp.concatenate([edge_index[0], jnp.zeros((npad,), jnp.int32)])
    dstp = jnp.concatenate(
        [edge_index[1], jnp.full((npad,), DST_SENTINEL, jnp.int32)])
    sum_pad, deg_pad = _sc_aggregate(feat, srcp, dstp)
    summed = sum_pad[:, :W_WIN].reshape(NW * W_WIN, D)[:N_NODES_K]
    deg = deg_pad[:, :W_WIN].reshape(NW * W_WIN)[:N_NODES_K]
    return _tc_combine(feat, summed, deg.reshape(N_NODES_K, 1),
                       W_self.T, W_neigh.T, b.reshape(1, D))


# speedup vs baseline: 1.4201x; 1.4201x over previous
import functools

import jax
import jax.numpy as jnp
from jax import lax
from jax.experimental import pallas as pl
from jax.experimental.pallas import tpu as pltpu
from jax.experimental.pallas import tpu_sc as plsc

N_NODES_K = 10000
N_EDGES_K = 160000
D = 256

NC = 2
NS = 16
NW = NC * NS
W_WIN = 313
ACC_ROWS = 320
TRASH = 316
SCAN = 1024
NSCAN = 158
NCH_PAD = 160
E_PAD = NCH_PAD * SCAN
G = 128
FLUSH_AT = 2048
CAP = 3328
DST_SENTINEL = 1 << 30

_sc_mesh = plsc.VectorSubcoreMesh(core_axis_name="c", subcore_axis_name="s")


@functools.partial(
    pl.kernel,
    out_type=[
        jax.ShapeDtypeStruct((NW, ACC_ROWS, D), jnp.float32),
        jax.ShapeDtypeStruct((NW, ACC_ROWS), jnp.float32),
    ],
    mesh=_sc_mesh,
    scratch_types=[
        pltpu.VMEM((ACC_ROWS, D), jnp.float32),
        pltpu.VMEM((G, D), jnp.float32),
        pltpu.VMEM((SCAN,), jnp.int32),
        pltpu.VMEM((SCAN,), jnp.int32),
        pltpu.VMEM((SCAN,), jnp.int32),
        pltpu.VMEM((SCAN,), jnp.int32),
        pltpu.VMEM((CAP,), jnp.int32),
        pltpu.VMEM((CAP,), jnp.int32),
        pltpu.VMEM((G,), jnp.int32),
        pltpu.VMEM((G,), jnp.float32),
        pltpu.VMEM((ACC_ROWS,), jnp.float32),
        pltpu.VMEM_SHARED((NS * ACC_ROWS,), jnp.float32),
        pltpu.SemaphoreType.DMA,
        pltpu.SemaphoreType.DMA,
        pltpu.SemaphoreType.DMA,
    ],
    compiler_params=pltpu.CompilerParams(needs_layout_passes=False),
)
def _sc_aggregate(feat_hbm, src_hbm, dst_hbm, sum_hbm, deg_hbm,
                  acc, rows, dstA, srcA, dstB, srcB,
                  csrc, cld, cidx, ones, degf, degsp,
                  semA, semB, semG):
    c = lax.axis_index("c")
    s = lax.axis_index("s")
    w = s * NC + c
    base = w * W_WIN
    dbase = s * ACC_ROWS
    zero16 = jnp.zeros((16,), jnp.float32)
    one16 = jnp.ones((16,), jnp.float32)
    trash16 = jnp.full((16,), TRASH, jnp.int32)
    zero16i = jnp.zeros((16,), jnp.int32)
    iota16 = lax.iota(jnp.int32, 16)

    def _zrow(i, carry):
        for j in range(D // 16):
            acc[i, pl.ds(j * 16, 16)] = zero16
        return carry
    lax.fori_loop(0, ACC_ROWS, _zrow, 0)
    for j in range(ACC_ROWS // 16):
        degf[pl.ds(j * 16, 16)] = zero16
    for j in range(G // 16):
        ones[pl.ds(j * 16, 16)] = one16
    def _zc(i, carry):
        csrc[pl.ds(i * 16, 16)] = zero16i
        return carry
    lax.fori_loop(0, CAP // 16, _zc, 0)
    pltpu.sync_copy(degf, degsp.at[pl.ds(dbase, ACC_ROWS)])

    def _wait_scan(sem, dbuf, sbuf):
        pltpu.make_async_copy(dst_hbm.at[pl.ds(0, SCAN)], dbuf, sem).wait()
        pltpu.make_async_copy(src_hbm.at[pl.ds(0, SCAN)], sbuf, sem).wait()

    def _flush(n):
        a0 = (n // 16) * 16
        keep = iota16 < (n - a0)
        csrc[pl.ds(a0, 16)] = jnp.where(keep, csrc[pl.ds(a0, 16)], zero16i)
        cld[pl.ds(a0, 16)] = jnp.where(keep, cld[pl.ds(a0, 16)], trash16)
        for t in range(1, G // 16):
            csrc[pl.ds(a0 + 16 * t, 16)] = zero16i
            cld[pl.ds(a0 + 16 * t, 16)] = trash16
        nch = (n + G - 1) // G

        def _gchunk(g, carry):
            p0 = g * G
            cp = pltpu.async_copy(feat_hbm.at[csrc.at[pl.ds(p0, G)]],
                                  rows, semG)
            for j in range(G // 16):
                cidx[pl.ds(j * 16, 16)] = cld[pl.ds(p0 + j * 16, 16)] + dbase
            pltpu.sync_copy(ones, degsp.at[cidx], add=True)
            cp.wait()

            def _edge(e, carry2):
                rv = cld[pl.ds(p0 + e, 16)]
                r = rv[0]
                for j in range(D // 16):
                    plsc.addupdate(acc.at[r, pl.ds(j * 16, 16)],
                                   rows[e, pl.ds(j * 16, 16)])
                return carry2
            lax.fori_loop(0, G, _edge, 0)
            return carry
        lax.fori_loop(0, nch, _gchunk, 0)

    def _proc(dbuf, sbuf, cnt):
        def _step(i, cnt2):
            d = dbuf[pl.ds(i * 16, 16)]
            ld = d - base
            m = plsc.bitcast(ld, jnp.uint32) < jnp.uint32(W_WIN)
            incl = plsc.cumsum(jnp.where(m, 1, 0).astype(jnp.int32))
            pos = cnt2 + incl - 1
            plsc.store_scatter(csrc, [pos], sbuf[pl.ds(i * 16, 16)], mask=m)
            plsc.store_scatter(cld, [pos], ld, mask=m)
            return cnt2 + jnp.max(incl)
        cnt = lax.fori_loop(0, SCAN // 16, _step, cnt)
        do_flush = cnt >= FLUSH_AT
        pl.when(do_flush)(lambda: _flush(cnt))
        return jnp.where(do_flush, 0, cnt)

    pltpu.async_copy(dst_hbm.at[pl.ds(0, SCAN)], dstA, semA)
    pltpu.async_copy(src_hbm.at[pl.ds(0, SCAN)], srcA, semA)

    def _scan2(h, cnt):
        offB = (2 * h + 1) * SCAN
        offA2 = (2 * h + 2) * SCAN
        pltpu.async_copy(dst_hbm.at[pl.ds(offB, SCAN)], dstB, semB)
        pltpu.async_copy(src_hbm.at[pl.ds(offB, SCAN)], srcB, semB)
        _wait_scan(semA, dstA, srcA)
        cnt = _proc(dstA, srcA, cnt)
        pltpu.async_copy(dst_hbm.at[pl.ds(offA2, SCAN)], dstA, semA)
        pltpu.async_copy(src_hbm.at[pl.ds(offA2, SCAN)], srcA, semA)
        _wait_scan(semB, dstB, srcB)
        cnt = _proc(dstB, srcB, cnt)
        return cnt
    cnt = lax.fori_loop(0, NSCAN // 2, _scan2, jnp.int32(0))
    _wait_scan(semA, dstA, srcA)
    pl.when(cnt > 0)(lambda: _flush(cnt))

    pltpu.sync_copy(acc, sum_hbm.at[w])
    pltpu.sync_copy(degsp.at[pl.ds(dbase, ACC_ROWS)], degf)
    pltpu.sync_copy(degf, deg_hbm.at[w])


def _tc_body(feat_ref, sum_ref, deg_ref, wst_ref, wnt_ref, b_ref, out_ref):
    rcp = 1.0 / jnp.maximum(deg_ref[...], 1.0)
    h = sum_ref[...] * rcp
    out_ref[...] = (
        jnp.dot(feat_ref[...], wst_ref[...], preferred_element_type=jnp.float32)
        + jnp.dot(h, wnt_ref[...], preferred_element_type=jnp.float32)
        + b_ref[...]
    )


_BLK = 200
_tc_combine = pl.pallas_call(
    _tc_body,
    grid=(N_NODES_K // _BLK,),
    in_specs=[
        pl.BlockSpec((_BLK, D), lambda i: (i, 0)),
        pl.BlockSpec((_BLK, D), lambda i: (i, 0)),
        pl.BlockSpec((_BLK, 1), lambda i: (i, 0)),
        pl.BlockSpec((D, D), lambda i: (0, 0)),
        pl.BlockSpec((D, D), lambda i: (0, 0)),
        pl.BlockSpec((1, D), lambda i: (0, 0)),
    ],
    out_specs=pl.BlockSpec((_BLK, D), lambda i: (i, 0)),
    out_shape=jax.ShapeDtypeStruct((N_NODES_K, D), jnp.float32),
)


@jax.jit
def kernel(feat, edge_index, W_self, W_neigh, b):
    npad = E_PAD - N_EDGES_K
    srcp = jnp.concatenate([edge_index[0], jnp.zeros((npad,), jnp.int32)])
    dstp = jnp.concatenate(
        [edge_index[1], jnp.full((npad,), DST_SENTINEL, jnp.int32)])
    sum_pad, deg_pad = _sc_aggregate(feat, srcp, dstp)
    summed = sum_pad[:, :W_WIN].reshape(NW * W_WIN, D)[:N_NODES_K]
    deg = deg_pad[:, :W_WIN].reshape(NW * W_WIN)[:N_NODES_K]
    return _tc_combine(feat, summed, deg.reshape(N_NODES_K, 1),
                       W_self.T, W_neigh.T, b.reshape(1, D))


# speedup vs baseline: 1.4413x; 1.0150x over previous
import functools

import jax
import jax.numpy as jnp
from jax import lax
from jax.experimental import pallas as pl
from jax.experimental.pallas import tpu as pltpu
from jax.experimental.pallas import tpu_sc as plsc

N_NODES_K = 10000
N_EDGES_K = 160000
D = 256

NC = 2
NS = 16
NW = NC * NS
W_WIN = 313
ACC_ROWS = 320
TRASH = 316
SCAN = 1024
NSCAN = 158
NCH_PAD = 160
E_PAD = NCH_PAD * SCAN
G = 64
FLUSH_AT = 2048
CAP = 3328
DST_SENTINEL = 1 << 30

_sc_mesh = plsc.VectorSubcoreMesh(core_axis_name="c", subcore_axis_name="s")


@functools.partial(
    pl.kernel,
    out_type=[
        jax.ShapeDtypeStruct((NW, ACC_ROWS, D), jnp.float32),
        jax.ShapeDtypeStruct((NW, ACC_ROWS), jnp.float32),
    ],
    mesh=_sc_mesh,
    scratch_types=[
        pltpu.VMEM((ACC_ROWS, D), jnp.float32),
        pltpu.VMEM((G, D), jnp.float32),
        pltpu.VMEM((G, D), jnp.float32),
        pltpu.VMEM((SCAN,), jnp.int32),
        pltpu.VMEM((SCAN,), jnp.int32),
        pltpu.VMEM((SCAN,), jnp.int32),
        pltpu.VMEM((SCAN,), jnp.int32),
        pltpu.VMEM((CAP,), jnp.int32),
        pltpu.VMEM((CAP,), jnp.int32),
        pltpu.VMEM((G,), jnp.int32),
        pltpu.VMEM((G,), jnp.float32),
        pltpu.VMEM((ACC_ROWS,), jnp.float32),
        pltpu.VMEM_SHARED((NS * ACC_ROWS,), jnp.float32),
        pltpu.SemaphoreType.DMA,
        pltpu.SemaphoreType.DMA,
        pltpu.SemaphoreType.DMA,
        pltpu.SemaphoreType.DMA,
    ],
    compiler_params=pltpu.CompilerParams(needs_layout_passes=False),
)
def _sc_aggregate(feat_hbm, src_hbm, dst_hbm, sum_hbm, deg_hbm,
                  acc, rowsA, rowsB, dstA, srcA, dstB, srcB,
                  csrc, cld, cidx, ones, degf, degsp,
                  semA, semB, semGA, semGB):
    c = lax.axis_index("c")
    s = lax.axis_index("s")
    w = s * NC + c
    base = w * W_WIN
    dbase = s * ACC_ROWS
    zero16 = jnp.zeros((16,), jnp.float32)
    one16 = jnp.ones((16,), jnp.float32)
    trash16 = jnp.full((16,), TRASH, jnp.int32)
    zero16i = jnp.zeros((16,), jnp.int32)
    iota16 = lax.iota(jnp.int32, 16)

    def _zrow(i, carry):
        for j in range(D // 16):
            acc[i, pl.ds(j * 16, 16)] = zero16
        return carry
    lax.fori_loop(0, ACC_ROWS, _zrow, 0)
    for j in range(ACC_ROWS // 16):
        degf[pl.ds(j * 16, 16)] = zero16
    for j in range(G // 16):
        ones[pl.ds(j * 16, 16)] = one16
    def _zc(i, carry):
        csrc[pl.ds(i * 16, 16)] = zero16i
        return carry
    lax.fori_loop(0, CAP // 16, _zc, 0)
    pltpu.sync_copy(degf, degsp.at[pl.ds(dbase, ACC_ROWS)])

    def _wait_scan(sem, dbuf, sbuf):
        pltpu.make_async_copy(dst_hbm.at[pl.ds(0, SCAN)], dbuf, sem).wait()
        pltpu.make_async_copy(src_hbm.at[pl.ds(0, SCAN)], sbuf, sem).wait()

    def _flush(n):
        a0 = (n // 16) * 16
        keep = iota16 < (n - a0)
        csrc[pl.ds(a0, 16)] = jnp.where(keep, csrc[pl.ds(a0, 16)], zero16i)
        cld[pl.ds(a0, 16)] = jnp.where(keep, cld[pl.ds(a0, 16)], trash16)
        for t in range(1, G // 16):
            csrc[pl.ds(a0 + 16 * t, 16)] = zero16i
            cld[pl.ds(a0 + 16 * t, 16)] = trash16
        nch = (n + G - 1) // G
        nh = (nch + 1) // 2

        def _chunk(p0, rbuf):
            for j in range(G // 16):
                cidx[pl.ds(j * 16, 16)] = cld[pl.ds(p0 + j * 16, 16)] + dbase
            pltpu.sync_copy(ones, degsp.at[cidx], add=True)

            def _edge(e, carry2):
                rv = cld[pl.ds(p0 + e, 16)]
                r = rv[0]
                for j in range(D // 16):
                    plsc.addupdate(acc.at[r, pl.ds(j * 16, 16)],
                                   rbuf[e, pl.ds(j * 16, 16)])
                return carry2
            lax.fori_loop(0, G, _edge, 0)

        def _pair(g2, carry):
            pA = (2 * g2) * G
            pB = pA + G
            cpA = pltpu.async_copy(feat_hbm.at[csrc.at[pl.ds(pA, G)]],
                                   rowsA, semGA)
            cpB = pltpu.async_copy(feat_hbm.at[csrc.at[pl.ds(pB, G)]],
                                   rowsB, semGB)
            cpA.wait()
            pl.when(2 * g2 < nch)(lambda: _chunk(pA, rowsA))
            cpB.wait()
            pl.when(2 * g2 + 1 < nch)(lambda: _chunk(pB, rowsB))
            return carry
        lax.fori_loop(0, nh, _pair, 0)

    def _proc(dbuf, sbuf, cnt):
        def _step(i, cnt2):
            d = dbuf[pl.ds(i * 16, 16)]
            ld = d - base
            m = plsc.bitcast(ld, jnp.uint32) < jnp.uint32(W_WIN)
            incl = plsc.cumsum(jnp.where(m, 1, 0).astype(jnp.int32))
            pos = cnt2 + incl - 1
            plsc.store_scatter(csrc, [pos], sbuf[pl.ds(i * 16, 16)], mask=m)
            plsc.store_scatter(cld, [pos], ld, mask=m)
            return cnt2 + jnp.max(incl)
        cnt = lax.fori_loop(0, SCAN // 16, _step, cnt)
        do_flush = cnt >= FLUSH_AT
        pl.when(do_flush)(lambda: _flush(cnt))
        return jnp.where(do_flush, 0, cnt)

    pltpu.async_copy(dst_hbm.at[pl.ds(0, SCAN)], dstA, semA)
    pltpu.async_copy(src_hbm.at[pl.ds(0, SCAN)], srcA, semA)

    def _scan2(h, cnt):
        offB = (2 * h + 1) * SCAN
        offA2 = (2 * h + 2) * SCAN
        pltpu.async_copy(dst_hbm.at[pl.ds(offB, SCAN)], dstB, semB)
        pltpu.async_copy(src_hbm.at[pl.ds(offB, SCAN)], srcB, semB)
        _wait_scan(semA, dstA, srcA)
        cnt = _proc(dstA, srcA, cnt)
        pltpu.async_copy(dst_hbm.at[pl.ds(offA2, SCAN)], dstA, semA)
        pltpu.async_copy(src_hbm.at[pl.ds(offA2, SCAN)], srcA, semA)
        _wait_scan(semB, dstB, srcB)
        cnt = _proc(dstB, srcB, cnt)
        return cnt
    cnt = lax.fori_loop(0, NSCAN // 2, _scan2, jnp.int32(0))
    _wait_scan(semA, dstA, srcA)
    pl.when(cnt > 0)(lambda: _flush(cnt))

    pltpu.sync_copy(acc, sum_hbm.at[w])
    pltpu.sync_copy(degsp.at[pl.ds(dbase, ACC_ROWS)], degf)
    pltpu.sync_copy(degf, deg_hbm.at[w])


def _tc_body(feat_ref, sum_ref, deg_ref, wst_ref, wnt_ref, b_ref, out_ref):
    rcp = 1.0 / jnp.maximum(deg_ref[...], 1.0)
    h = sum_ref[...] * rcp
    out_ref[...] = (
        jnp.dot(feat_ref[...], wst_ref[...], preferred_element_type=jnp.float32)
        + jnp.dot(h, wnt_ref[...], preferred_element_type=jnp.float32)
        + b_ref[...]
    )


_BLK = 200
_tc_combine = pl.pallas_call(
    _tc_body,
    grid=(N_NODES_K // _BLK,),
    in_specs=[
        pl.BlockSpec((_BLK, D), lambda i: (i, 0)),
        pl.BlockSpec((_BLK, D), lambda i: (i, 0)),
        pl.BlockSpec((_BLK, 1), lambda i: (i, 0)),
        pl.BlockSpec((D, D), lambda i: (0, 0)),
        pl.BlockSpec((D, D), lambda i: (0, 0)),
        pl.BlockSpec((1, D), lambda i: (0, 0)),
    ],
    out_specs=pl.BlockSpec((_BLK, D), lambda i: (i, 0)),
    out_shape=jax.ShapeDtypeStruct((N_NODES_K, D), jnp.float32),
)


@jax.jit
def kernel(feat, edge_index, W_self, W_neigh, b):
    npad = E_PAD - N_EDGES_K
    srcp = jnp.concatenate([edge_index[0], jnp.zeros((npad,), jnp.int32)])
    dstp = jnp.concatenate(
        [edge_index[1], jnp.full((npad,), DST_SENTINEL, jnp.int32)])
    sum_pad, deg_pad = _sc_aggregate(feat, srcp, dstp)
    summed = sum_pad[:, :W_WIN].reshape(NW * W_WIN, D)[:N_NODES_K]
    deg = deg_pad[:, :W_WIN].reshape(NW * W_WIN)[:N_NODES_K]
    return _tc_combine(feat, summed, deg.reshape(N_NODES_K, 1),
                       W_self.T, W_neigh.T, b.reshape(1, D))


# speedup vs baseline: 1.4580x; 1.0116x over previous
import functools

import jax
import jax.numpy as jnp
from jax import lax
from jax.experimental import pallas as pl
from jax.experimental.pallas import tpu as pltpu
from jax.experimental.pallas import tpu_sc as plsc

N_NODES_K = 10000
N_EDGES_K = 160000
D = 256

NC = 2
NS = 16
NW = NC * NS
W_WIN = 313
ACC_ROWS = 320
TRASH = 316
SCAN = 1024
NSCAN = 158
NCH_PAD = 160
E_PAD = NCH_PAD * SCAN
G = 64
FLUSH_AT = 2048
CAP = 3328
DST_SENTINEL = 1 << 30

_sc_mesh = plsc.VectorSubcoreMesh(core_axis_name="c", subcore_axis_name="s")


@functools.partial(
    pl.kernel,
    out_type=[
        jax.ShapeDtypeStruct((NW, ACC_ROWS, D), jnp.float32),
        jax.ShapeDtypeStruct((NW, ACC_ROWS), jnp.float32),
    ],
    mesh=_sc_mesh,
    scratch_types=[
        pltpu.VMEM((ACC_ROWS, D), jnp.float32),
        pltpu.VMEM((G, D), jnp.float32),
        pltpu.VMEM((G, D), jnp.float32),
        pltpu.VMEM((SCAN,), jnp.int32),
        pltpu.VMEM((SCAN,), jnp.int32),
        pltpu.VMEM((SCAN,), jnp.int32),
        pltpu.VMEM((SCAN,), jnp.int32),
        pltpu.VMEM((CAP,), jnp.int32),
        pltpu.VMEM((CAP,), jnp.int32),
        pltpu.VMEM((G,), jnp.int32),
        pltpu.VMEM((G,), jnp.float32),
        pltpu.VMEM((ACC_ROWS,), jnp.float32),
        pltpu.VMEM_SHARED((NS * ACC_ROWS,), jnp.float32),
        pltpu.SemaphoreType.DMA,
        pltpu.SemaphoreType.DMA,
        pltpu.SemaphoreType.DMA,
        pltpu.SemaphoreType.DMA,
    ],
    compiler_params=pltpu.CompilerParams(needs_layout_passes=False),
)
def _sc_aggregate(feat_hbm, src_hbm, dst_hbm, sum_hbm, deg_hbm,
                  acc, rowsA, rowsB, dstA, srcA, dstB, srcB,
                  csrc, cld, cidx, ones, degf, degsp,
                  semA, semB, semGA, semGB):
    c = lax.axis_index("c")
    s = lax.axis_index("s")
    w = s * NC + c
    base = w * W_WIN
    dbase = s * ACC_ROWS
    zero16 = jnp.zeros((16,), jnp.float32)
    one16 = jnp.ones((16,), jnp.float32)
    trash16 = jnp.full((16,), TRASH, jnp.int32)
    zero16i = jnp.zeros((16,), jnp.int32)
    iota16 = lax.iota(jnp.int32, 16)

    def _zrow(i, carry):
        for j in range(D // 16):
            acc[i, pl.ds(j * 16, 16)] = zero16
        return carry
    lax.fori_loop(0, ACC_ROWS, _zrow, 0)
    for j in range(ACC_ROWS // 16):
        degf[pl.ds(j * 16, 16)] = zero16
    for j in range(G // 16):
        ones[pl.ds(j * 16, 16)] = one16
    def _zc(i, carry):
        csrc[pl.ds(i * 16, 16)] = zero16i
        return carry
    lax.fori_loop(0, CAP // 16, _zc, 0)
    pltpu.sync_copy(degf, degsp.at[pl.ds(dbase, ACC_ROWS)])

    def _wait_scan(sem, dbuf, sbuf):
        pltpu.make_async_copy(dst_hbm.at[pl.ds(0, SCAN)], dbuf, sem).wait()
        pltpu.make_async_copy(src_hbm.at[pl.ds(0, SCAN)], sbuf, sem).wait()

    def _flush(n):
        a0 = (n // 16) * 16
        keep = iota16 < (n - a0)
        csrc[pl.ds(a0, 16)] = jnp.where(keep, csrc[pl.ds(a0, 16)], zero16i)
        cld[pl.ds(a0, 16)] = jnp.where(keep, cld[pl.ds(a0, 16)], trash16)
        for t in range(1, G // 16):
            csrc[pl.ds(a0 + 16 * t, 16)] = zero16i
            cld[pl.ds(a0 + 16 * t, 16)] = trash16
        nch = (n + G - 1) // G
        nh = (nch + 1) // 2

        def _chunk(p0, rbuf):
            for j in range(G // 16):
                cidx[pl.ds(j * 16, 16)] = cld[pl.ds(p0 + j * 16, 16)] + dbase
            pltpu.sync_copy(ones, degsp.at[cidx], add=True)

            def _edge(e, carry2):
                rv = cld[pl.ds(p0 + e, 16)]
                r = rv[0]
                for j in range(D // 16):
                    plsc.addupdate(acc.at[r, pl.ds(j * 16, 16)],
                                   rbuf[e, pl.ds(j * 16, 16)])
                return carry2
            lax.fori_loop(0, G, _edge, 0)

        def _pair(g2, carry):
            pA = (2 * g2) * G
            pB = pA + G
            cpA = pltpu.async_copy(feat_hbm.at[csrc.at[pl.ds(pA, G)]],
                                   rowsA, semGA)
            cpB = pltpu.async_copy(feat_hbm.at[csrc.at[pl.ds(pB, G)]],
                                   rowsB, semGB)
            cpA.wait()
            pl.when(2 * g2 < nch)(lambda: _chunk(pA, rowsA))
            cpB.wait()
            pl.when(2 * g2 + 1 < nch)(lambda: _chunk(pB, rowsB))
            return carry
        lax.fori_loop(0, nh, _pair, 0)

    def _proc(dbuf, sbuf, cnt):
        def _step(i, cnt2):
            d = dbuf[pl.ds(i * 16, 16)]
            ld = d - base
            m = plsc.bitcast(ld, jnp.uint32) < jnp.uint32(W_WIN)
            incl = plsc.cumsum(jnp.where(m, 1, 0).astype(jnp.int32))
            pos = cnt2 + incl - 1
            plsc.store_scatter(csrc, [pos], sbuf[pl.ds(i * 16, 16)], mask=m)
            plsc.store_scatter(cld, [pos], ld, mask=m)
            return cnt2 + jnp.max(incl)
        cnt = lax.fori_loop(0, SCAN // 16, _step, cnt)
        do_flush = cnt >= FLUSH_AT
        pl.when(do_flush)(lambda: _flush(cnt))
        return jnp.where(do_flush, 0, cnt)

    pltpu.async_copy(dst_hbm.at[pl.ds(0, SCAN)], dstA, semA)
    pltpu.async_copy(src_hbm.at[pl.ds(0, SCAN)], srcA, semA)

    def _scan2(h, cnt):
        offB = (2 * h + 1) * SCAN
        offA2 = (2 * h + 2) * SCAN
        pltpu.async_copy(dst_hbm.at[pl.ds(offB, SCAN)], dstB, semB)
        pltpu.async_copy(src_hbm.at[pl.ds(offB, SCAN)], srcB, semB)
        _wait_scan(semA, dstA, srcA)
        cnt = _proc(dstA, srcA, cnt)
        pltpu.async_copy(dst_hbm.at[pl.ds(offA2, SCAN)], dstA, semA)
        pltpu.async_copy(src_hbm.at[pl.ds(offA2, SCAN)], srcA, semA)
        _wait_scan(semB, dstB, srcB)
        cnt = _proc(dstB, srcB, cnt)
        return cnt
    cnt = lax.fori_loop(0, NSCAN // 2, _scan2, jnp.int32(0))
    _wait_scan(semA, dstA, srcA)
    pl.when(cnt > 0)(lambda: _flush(cnt))

    pltpu.sync_copy(acc, sum_hbm.at[w])
    pltpu.sync_copy(degsp.at[pl.ds(dbase, ACC_ROWS)], degf)
    pltpu.sync_copy(degf, deg_hbm.at[w])


def _tc_self_body(feat_ref, wst_ref, b_ref, out_ref):
    out_ref[...] = (
        jnp.dot(feat_ref[...], wst_ref[...], preferred_element_type=jnp.float32)
        + b_ref[...]
    )


def _tc_neigh_body(self_ref, sum_ref, deg_ref, wnt_ref, out_ref):
    rcp = 1.0 / jnp.maximum(deg_ref[...], 1.0)
    h = sum_ref[...] * rcp
    out_ref[...] = self_ref[...] + jnp.dot(
        h, wnt_ref[...], preferred_element_type=jnp.float32)


_BLK = 200
_tc_self = pl.pallas_call(
    _tc_self_body,
    grid=(N_NODES_K // _BLK,),
    in_specs=[
        pl.BlockSpec((_BLK, D), lambda i: (i, 0)),
        pl.BlockSpec((D, D), lambda i: (0, 0)),
        pl.BlockSpec((1, D), lambda i: (0, 0)),
    ],
    out_specs=pl.BlockSpec((_BLK, D), lambda i: (i, 0)),
    out_shape=jax.ShapeDtypeStruct((N_NODES_K, D), jnp.float32),
)
_tc_neigh = pl.pallas_call(
    _tc_neigh_body,
    grid=(N_NODES_K // _BLK,),
    in_specs=[
        pl.BlockSpec((_BLK, D), lambda i: (i, 0)),
        pl.BlockSpec((_BLK, D), lambda i: (i, 0)),
        pl.BlockSpec((_BLK, 1), lambda i: (i, 0)),
        pl.BlockSpec((D, D), lambda i: (0, 0)),
    ],
    out_specs=pl.BlockSpec((_BLK, D), lambda i: (i, 0)),
    out_shape=jax.ShapeDtypeStruct((N_NODES_K, D), jnp.float32),
)


@jax.jit
def kernel(feat, edge_index, W_self, W_neigh, b):
    npad = E_PAD - N_EDGES_K
    srcp = jnp.concatenate([edge_index[0], jnp.zeros((npad,), jnp.int32)])
    dstp = jnp.concatenate(
        [edge_index[1], jnp.full((npad,), DST_SENTINEL, jnp.int32)])
    sum_pad, deg_pad = _sc_aggregate(feat, srcp, dstp)
    self_part = _tc_self(feat, W_self.T, b.reshape(1, D))
    summed = sum_pad[:, :W_WIN].reshape(NW * W_WIN, D)[:N_NODES_K]
    deg = deg_pad[:, :W_WIN].reshape(NW * W_WIN)[:N_NODES_K]
    return _tc_neigh(self_part, summed, deg.reshape(N_NODES_K, 1), W_neigh.T)
